# baseline (speedup 1.0000x reference)
import jax
import jax.numpy as jnp
from jax import lax
from jax.experimental import pallas as pl
from jax.experimental.pallas import tpu as pltpu

_IN = 768
_E = 256


def _fused_body(x_ref, w_emb_ref, w_a1_ref, b_a1_ref, w_a2_ref,
                w_num_ref, r2_ref, b_tail_ref, out_ref):
    tb = x_ref.shape[0]
    half = tb // 2
    contract = (((1,), (1,)), ((), ()))
    for i in range(2):
        sl = pl.ds(i * half, half)
        x = x_ref[sl, :]
        h0 = jnp.dot(x, w_emb_ref[...],
                     preferred_element_type=jnp.float32)
        t = jnp.dot(h0, w_a1_ref[...],
                    preferred_element_type=jnp.float32) + b_a1_ref[...]
        t = jnp.maximum(t, 0.0)
        a0 = jnp.dot(t, w_a2_ref[...],
                     preferred_element_type=jnp.float32)
        e0 = jnp.exp2(a0)
        num = jnp.dot(e0 * h0, w_num_ref[...],
                      preferred_element_type=jnp.float32)
        dc = jnp.dot(e0, r2_ref[...],
                     preferred_element_type=jnp.float32)
        out_ref[sl, :] = ((num + dc[:, 1:2])
                          * pl.reciprocal(dc[:, 0:1], approx=True)
                          + b_tail_ref[...])


def kernel(x, w_emb, b_emb, w_a1, b_a1, w_a2, b_a2, w_fc, b_fc, w_out, b_out):
    B, IN = x.shape
    assert IN == _IN

    w_tail = (w_fc @ w_out).reshape(1, _E)
    b_tail = (b_fc @ w_out + b_out).reshape(1, 1)
    f = jnp.exp(b_a2.reshape(1, _E))
    w_num = (w_tail * f).reshape(_E, 1)
    r2 = jnp.concatenate([f, f * b_emb.reshape(1, _E) * w_tail], axis=0).T
    b_a1f = b_emb @ w_a1 + b_a1
    w_a2s = w_a2 * jnp.float32(1.4426950408889634)

    Bp = ((B + 255) // 256) * 256
    tb = next(t for t in (2048, 1024, 512, 256) if Bp % t == 0)
    if Bp != B:
        x = jnp.pad(x, ((0, Bp - B), (0, 0)))

    full = lambda shape: pl.BlockSpec(shape, lambda i: (0, 0))
    out = pl.pallas_call(
        _fused_body,
        out_shape=jax.ShapeDtypeStruct((Bp, 1), jnp.float32),
        grid=(Bp // tb,),
        in_specs=[
            pl.BlockSpec((tb, IN), lambda i: (i, 0)),
            full((IN, _E)),
            full((_E, _E)), full((1, _E)),
            full((_E, _E)),
            full((_E, 1)), full((_E, 2)), full((1, 1)),
        ],
        out_specs=pl.BlockSpec((tb, 1), lambda i: (i, 0)),
        compiler_params=pltpu.CompilerParams(
            dimension_semantics=("parallel",)),
    )(x, w_emb, w_a1, b_a1f, w_a2s, w_num, r2, b_tail)

    return out[:B]


# speedup vs baseline: 1.1846x; 1.1846x over previous
import jax
import jax.numpy as jnp
from jax import lax
from jax.experimental import pallas as pl
from jax.experimental.pallas import tpu as pltpu

_IN = 768
_E = 256


def _fused_body(x_ref, w_emb_ref, w_a1_ref, b_a1_ref, w_a2_ref,
                w_num_ref, r2_ref, b_tail_ref, out_ref):
    contract = (((1,), (1,)), ((), ()))
    x = x_ref[...]
    h0 = jnp.dot(x, w_emb_ref[...],
                 preferred_element_type=jnp.float32)
    t = jnp.dot(h0, w_a1_ref[...],
                preferred_element_type=jnp.float32) + b_a1_ref[...]
    t = jnp.maximum(t, 0.0)
    a0 = jnp.dot(t, w_a2_ref[...],
                 preferred_element_type=jnp.float32)
    e0 = jnp.exp2(a0)
    num = lax.dot_general(w_num_ref[...], e0 * h0, contract,
                          preferred_element_type=jnp.float32)
    dc = lax.dot_general(r2_ref[...], e0, contract,
                         preferred_element_type=jnp.float32)
    out_ref[...] = ((num + dc[1:2, :])
                    * pl.reciprocal(dc[0:1, :], approx=True)
                    + b_tail_ref[...])


def kernel(x, w_emb, b_emb, w_a1, b_a1, w_a2, b_a2, w_fc, b_fc, w_out, b_out):
    B, IN = x.shape
    assert IN == _IN

    w_tail = (w_fc @ w_out).reshape(1, _E)
    b_tail = (b_fc @ w_out + b_out).reshape(1, 1)
    f = jnp.exp(b_a2.reshape(1, _E))
    w_num = w_tail * f
    r2 = jnp.concatenate([f, f * b_emb.reshape(1, _E) * w_tail], axis=0)
    b_a1f = b_emb @ w_a1 + b_a1
    w_a2s = w_a2 * jnp.float32(1.4426950408889634)

    Bp = ((B + 255) // 256) * 256
    tb = next(t for t in (2048, 1024, 512, 256) if Bp % t == 0)
    if Bp != B:
        x = jnp.pad(x, ((0, Bp - B), (0, 0)))

    full = lambda shape: pl.BlockSpec(shape, lambda i: (0, 0))
    out = pl.pallas_call(
        _fused_body,
        out_shape=jax.ShapeDtypeStruct((1, Bp), jnp.float32),
        grid=(Bp // tb,),
        in_specs=[
            pl.BlockSpec((tb, IN), lambda i: (i, 0)),
            full((IN, _E)),
            full((_E, _E)), full((1, _E)),
            full((_E, _E)),
            full((1, _E)), full((2, _E)), full((1, 1)),
        ],
        out_specs=pl.BlockSpec((1, tb), lambda i: (0, i)),
        compiler_params=pltpu.CompilerParams(
            dimension_semantics=("parallel",)),
    )(x, w_emb, w_a1, b_a1f, w_a2s, w_num, r2, b_tail)

    return out.reshape(Bp, 1)[:B]


# speedup vs baseline: 1.3961x; 1.1785x over previous
import jax
import jax.numpy as jnp
from jax import lax
from jax.experimental import pallas as pl
from jax.experimental.pallas import tpu as pltpu

_IN = 768
_E = 256


def _fused_body(x_ref, w_emb_ref, b_emb_ref, w_a1_ref, b_a1_ref,
                w_a2_ref, b_a2_ref, w_fc_ref, b_fc_ref, w_out_ref, b_out_ref,
                out_ref):
    c0 = (((0,), (1,)), ((), ()))
    w_tail = lax.dot_general(w_out_ref[...], w_fc_ref[...], c0,
                             preferred_element_type=jnp.float32)
    b_tail = jnp.dot(b_fc_ref[...], w_out_ref[...],
                     preferred_element_type=jnp.float32) + b_out_ref[...]

    x = x_ref[...]
    h = jnp.dot(x, w_emb_ref[...],
                preferred_element_type=jnp.float32) + b_emb_ref[...]
    t = jnp.dot(h, w_a1_ref[...],
                preferred_element_type=jnp.float32) + b_a1_ref[...]
    t = jnp.maximum(t, 0.0)
    a = jnp.dot(t, w_a2_ref[...],
                preferred_element_type=jnp.float32) + b_a2_ref[...]
    e = jnp.exp(a)
    contract = (((1,), (1,)), ((), ()))
    num = lax.dot_general(w_tail, e * h, contract,
                          preferred_element_type=jnp.float32)
    den = lax.dot_general(jnp.ones((1, _E), jnp.float32), e, contract,
                          preferred_element_type=jnp.float32)
    out_ref[...] = num * pl.reciprocal(den, approx=True) + b_tail


def kernel(x, w_emb, b_emb, w_a1, b_a1, w_a2, b_a2, w_fc, b_fc, w_out, b_out):
    B, IN = x.shape
    assert IN == _IN

    Bp = ((B + 255) // 256) * 256
    tb = next(t for t in (2048, 1024, 512, 256) if Bp % t == 0)
    if Bp != B:
        x = jnp.pad(x, ((0, Bp - B), (0, 0)))

    full = lambda shape: pl.BlockSpec(shape, lambda i: (0, 0))
    out = pl.pallas_call(
        _fused_body,
        out_shape=jax.ShapeDtypeStruct((1, Bp), jnp.float32),
        grid=(Bp // tb,),
        in_specs=[
            pl.BlockSpec((tb, IN), lambda i: (i, 0)),
            full((IN, _E)), full((1, _E)),
            full((_E, _E)), full((1, _E)),
            full((_E, _E)), full((1, _E)),
            full((_E, _E)), full((1, _E)),
            full((_E, 1)), full((1, 1)),
        ],
        out_specs=pl.BlockSpec((1, tb), lambda i: (0, i)),
        compiler_params=pltpu.CompilerParams(
            dimension_semantics=("parallel",)),
    )(x, w_emb, b_emb, w_a1, b_a1, w_a2, b_a2, w_fc, b_fc, w_out, b_out)

    return out.reshape(Bp, 1)[:B]
